# Initial kernel scaffold; baseline (speedup 1.0000x reference)
import jax
import jax.numpy as jnp
from jax.experimental import pallas as pl


def kernel(inputs_embeds, attention_mask, discourse_adj, coeff, bases, bias):
    raise NotImplementedError("write your pallas kernel here")


# speedup vs baseline: 33.3873x; 33.3873x over previous
import jax
import jax.numpy as jnp
from jax.experimental import pallas as pl
from jax.experimental.pallas import tpu as pltpu

_B, _N, _D = 4, 256, 512
_R, _L = 17, 3
_NR = _R - 1
_CH = 32768
_NCH = (_D * _D) // _CH


_PREC = jax.lax.Precision.HIGHEST


def _w_kernel(coeff_ref, bases_ref, w_ref):
    w_ref[0] = jnp.dot(coeff_ref[0], bases_ref[0], precision=_PREC,
                       preferred_element_type=jnp.float32)


def _msg_kernel(adjT_ref, h0_ref, w_ref, bias_ref, out_ref, h_s, acc):
    bi = pl.program_id(0)
    l = pl.program_id(1)
    r = pl.program_id(2)

    @pl.when((l == 0) & (r == 0))
    def _():
        h_s[...] = h0_ref[bi]

    @pl.when(r == 0)
    def _():
        acc[...] = jnp.zeros_like(acc[...])

    proj = jnp.dot(h_s[...], w_ref[0, 0], precision=_PREC,
                   preferred_element_type=jnp.float32)
    mt = (adjT_ref[bi] == (r + 1)).astype(jnp.float32)
    acc[...] += jnp.dot(mt, proj, precision=_PREC,
                        preferred_element_type=jnp.float32)

    @pl.when(r == _NR - 1)
    def _():
        h_s[...] = acc[...] + bias_ref[l][None, :]

    @pl.when((l == _L - 1) & (r == _NR - 1))
    def _():
        out_ref[0] = acc[...] + bias_ref[l][None, :]


def kernel(inputs_embeds, attention_mask, discourse_adj, coeff, bases, bias):
    del attention_mask

    bases_flat = bases.reshape(_L, _R, _D * _D)
    w_flat = pl.pallas_call(
        _w_kernel,
        grid=(_L, _NCH),
        in_specs=[
            pl.BlockSpec((1, _R, _R), lambda l, c: (l, 0, 0)),
            pl.BlockSpec((1, _R, _CH), lambda l, c: (l, 0, c)),
        ],
        out_specs=pl.BlockSpec((1, _R, _CH), lambda l, c: (l, 0, c)),
        out_shape=jax.ShapeDtypeStruct((_L, _R, _D * _D), jnp.float32),
    )(coeff, bases_flat)
    w = w_flat.reshape(_L, _R, _D, _D)

    adjT = jnp.swapaxes(discourse_adj, 1, 2)

    out = pl.pallas_call(
        _msg_kernel,
        grid=(_B, _L, _NR),
        in_specs=[
            pl.BlockSpec((_B, _N, _N), lambda bi, l, r: (0, 0, 0)),
            pl.BlockSpec((_B, _N, _D), lambda bi, l, r: (0, 0, 0)),
            pl.BlockSpec((1, 1, _D, _D), lambda bi, l, r: (l, r + 1, 0, 0)),
            pl.BlockSpec((_L, _D), lambda bi, l, r: (0, 0)),
        ],
        out_specs=pl.BlockSpec((1, _N, _D), lambda bi, l, r: (bi, 0, 0)),
        out_shape=jax.ShapeDtypeStruct((_B, _N, _D), jnp.float32),
        scratch_shapes=[
            pltpu.VMEM((_N, _D), jnp.float32),
            pltpu.VMEM((_N, _D), jnp.float32),
        ],
    )(adjT, inputs_embeds, w, bias)
    return out


# speedup vs baseline: 37.1241x; 1.1119x over previous
import jax
import jax.numpy as jnp
from jax.experimental import pallas as pl
from jax.experimental.pallas import tpu as pltpu

_B, _N, _D = 4, 256, 512
_R, _L = 17, 3
_NR = _R - 1
_CH = 32768
_NCH = (_D * _D) // _CH

_F32 = jnp.float32
_BF16 = jnp.bfloat16


def _split(x):
    hi = x.astype(_BF16)
    lo = (x - hi.astype(_F32)).astype(_BF16)
    return hi, lo


def _bdot(a, b):
    return jnp.dot(a, b, preferred_element_type=_F32)


def _w_kernel(coeff_ref, bases_ref, whi_ref, wlo_ref):
    w = jnp.dot(coeff_ref[0], bases_ref[0],
                precision=jax.lax.Precision.HIGHEST,
                preferred_element_type=_F32)
    hi, lo = _split(w)
    whi_ref[0] = hi
    wlo_ref[0] = lo


def _msg_kernel(adjT_ref, h0_ref, whi_ref, wlo_ref, bias_ref, out_ref,
                h_s, hhi_s, hlo_s, acc):
    bi = pl.program_id(0)
    l = pl.program_id(1)
    r = pl.program_id(2)

    @pl.when((l == 0) & (r == 0))
    def _():
        h_s[...] = h0_ref[bi]

    @pl.when(r == 0)
    def _():
        hi, lo = _split(h_s[...])
        hhi_s[...] = hi
        hlo_s[...] = lo
        acc[...] = jnp.zeros_like(acc[...])

    whi = whi_ref[0, 0]
    wlo = wlo_ref[0, 0]
    proj = (_bdot(hhi_s[...], whi) + _bdot(hlo_s[...], whi)
            + _bdot(hhi_s[...], wlo))
    phi, plo = _split(proj)
    mt = (adjT_ref[bi] == (r + 1)).astype(_BF16)
    acc[...] += _bdot(mt, phi) + _bdot(mt, plo)

    @pl.when(r == _NR - 1)
    def _():
        h_s[...] = acc[...] + bias_ref[l][None, :]

    @pl.when((l == _L - 1) & (r == _NR - 1))
    def _():
        out_ref[0] = acc[...] + bias_ref[l][None, :]


def kernel(inputs_embeds, attention_mask, discourse_adj, coeff, bases, bias):
    del attention_mask

    bases_flat = bases.reshape(_L, _R, _D * _D)
    w_hi, w_lo = pl.pallas_call(
        _w_kernel,
        grid=(_L, _NCH),
        in_specs=[
            pl.BlockSpec((1, _R, _R), lambda l, c: (l, 0, 0)),
            pl.BlockSpec((1, _R, _CH), lambda l, c: (l, 0, c)),
        ],
        out_specs=[
            pl.BlockSpec((1, _R, _CH), lambda l, c: (l, 0, c)),
            pl.BlockSpec((1, _R, _CH), lambda l, c: (l, 0, c)),
        ],
        out_shape=[
            jax.ShapeDtypeStruct((_L, _R, _D * _D), _BF16),
            jax.ShapeDtypeStruct((_L, _R, _D * _D), _BF16),
        ],
    )(coeff, bases_flat)
    w_hi = w_hi.reshape(_L, _R, _D, _D)
    w_lo = w_lo.reshape(_L, _R, _D, _D)

    adjT = jnp.swapaxes(discourse_adj, 1, 2)

    out = pl.pallas_call(
        _msg_kernel,
        grid=(_B, _L, _NR),
        in_specs=[
            pl.BlockSpec((_B, _N, _N), lambda bi, l, r: (0, 0, 0)),
            pl.BlockSpec((_B, _N, _D), lambda bi, l, r: (0, 0, 0)),
            pl.BlockSpec((1, 1, _D, _D), lambda bi, l, r: (l, r + 1, 0, 0)),
            pl.BlockSpec((1, 1, _D, _D), lambda bi, l, r: (l, r + 1, 0, 0)),
            pl.BlockSpec((_L, _D), lambda bi, l, r: (0, 0)),
        ],
        out_specs=pl.BlockSpec((1, _N, _D), lambda bi, l, r: (bi, 0, 0)),
        out_shape=jax.ShapeDtypeStruct((_B, _N, _D), _F32),
        scratch_shapes=[
            pltpu.VMEM((_N, _D), _F32),
            pltpu.VMEM((_N, _D), _BF16),
            pltpu.VMEM((_N, _D), _BF16),
            pltpu.VMEM((_N, _D), _F32),
        ],
    )(adjT, inputs_embeds, w_hi, w_lo, bias)
    return out


# speedup vs baseline: 37.2928x; 1.0045x over previous
import jax
import jax.numpy as jnp
from jax.experimental import pallas as pl
from jax.experimental.pallas import tpu as pltpu

_B, _N, _D = 4, 256, 512
_R, _L = 17, 3
_NR = _R - 1
_CH = 32768
_NCH = (_D * _D) // _CH

_F32 = jnp.float32
_BF16 = jnp.bfloat16


def _split(x):
    hi = x.astype(_BF16)
    lo = (x - hi.astype(_F32)).astype(_BF16)
    return hi, lo


def _bdot(a, b):
    return jnp.dot(a, b, preferred_element_type=_F32)


def _w_kernel(coeff_ref, bases_ref, whi_ref, wlo_ref):
    w = jnp.dot(coeff_ref[0], bases_ref[0],
                precision=jax.lax.Precision.HIGHEST,
                preferred_element_type=_F32)
    hi, lo = _split(w)
    whi_ref[0] = hi
    wlo_ref[0] = lo


def _msg_kernel(adjT_ref, h0_ref, whi_ref, wlo_ref, bias_ref, out_ref,
                h_s, hcat_s, adj2_s, pcat_s, acc):
    bi = pl.program_id(0)
    l = pl.program_id(1)
    r = pl.program_id(2)

    @pl.when((l == 0) & (r == 0))
    def _():
        h_s[...] = h0_ref[bi]
        a = adjT_ref[bi]
        adj2_s[:, :_N] = a
        adj2_s[:, _N:] = a

    @pl.when(r == 0)
    def _():
        hi, lo = _split(h_s[...])
        hcat_s[:_N, :] = hi
        hcat_s[_N:, :] = lo
        acc[...] = jnp.zeros_like(acc[...])

    t = _bdot(hcat_s[...], whi_ref[0, 0])
    proj = t[:_N] + t[_N:] + _bdot(hcat_s[:_N, :], wlo_ref[0, 0])
    phi, plo = _split(proj)
    pcat_s[:_N, :] = phi
    pcat_s[_N:, :] = plo
    mtc = (adj2_s[...] == (r + 1)).astype(_BF16)
    acc[...] += _bdot(mtc, pcat_s[...])

    @pl.when(r == _NR - 1)
    def _():
        h_s[...] = acc[...] + bias_ref[l][None, :]

    @pl.when((l == _L - 1) & (r == _NR - 1))
    def _():
        out_ref[0] = acc[...] + bias_ref[l][None, :]


def kernel(inputs_embeds, attention_mask, discourse_adj, coeff, bases, bias):
    del attention_mask

    bases_flat = bases.reshape(_L, _R, _D * _D)
    w_hi, w_lo = pl.pallas_call(
        _w_kernel,
        grid=(_L, _NCH),
        in_specs=[
            pl.BlockSpec((1, _R, _R), lambda l, c: (l, 0, 0)),
            pl.BlockSpec((1, _R, _CH), lambda l, c: (l, 0, c)),
        ],
        out_specs=[
            pl.BlockSpec((1, _R, _CH), lambda l, c: (l, 0, c)),
            pl.BlockSpec((1, _R, _CH), lambda l, c: (l, 0, c)),
        ],
        out_shape=[
            jax.ShapeDtypeStruct((_L, _R, _D * _D), _BF16),
            jax.ShapeDtypeStruct((_L, _R, _D * _D), _BF16),
        ],
    )(coeff, bases_flat)
    w_hi = w_hi.reshape(_L, _R, _D, _D)
    w_lo = w_lo.reshape(_L, _R, _D, _D)

    adjT = jnp.swapaxes(discourse_adj, 1, 2)

    out = pl.pallas_call(
        _msg_kernel,
        grid=(_B, _L, _NR),
        in_specs=[
            pl.BlockSpec((_B, _N, _N), lambda bi, l, r: (0, 0, 0)),
            pl.BlockSpec((_B, _N, _D), lambda bi, l, r: (0, 0, 0)),
            pl.BlockSpec((1, 1, _D, _D), lambda bi, l, r: (l, r + 1, 0, 0)),
            pl.BlockSpec((1, 1, _D, _D), lambda bi, l, r: (l, r + 1, 0, 0)),
            pl.BlockSpec((_L, _D), lambda bi, l, r: (0, 0)),
        ],
        out_specs=pl.BlockSpec((1, _N, _D), lambda bi, l, r: (bi, 0, 0)),
        out_shape=jax.ShapeDtypeStruct((_B, _N, _D), _F32),
        scratch_shapes=[
            pltpu.VMEM((_N, _D), _F32),
            pltpu.VMEM((2 * _N, _D), _BF16),
            pltpu.VMEM((_N, 2 * _N), jnp.int32),
            pltpu.VMEM((2 * _N, _D), _BF16),
            pltpu.VMEM((_N, _D), _F32),
        ],
    )(adjT, inputs_embeds, w_hi, w_lo, bias)
    return out


# speedup vs baseline: 70.9487x; 1.9025x over previous
import jax
import jax.numpy as jnp
from jax.experimental import pallas as pl
from jax.experimental.pallas import tpu as pltpu

_B, _N, _D = 4, 256, 512
_R, _L = 17, 3
_NR = _R - 1

_F32 = jnp.float32
_BF16 = jnp.bfloat16


def _split(x):
    hi = x.astype(_BF16)
    lo = (x - hi.astype(_F32)).astype(_BF16)
    return hi, lo


def _bdot(a, b):
    return jnp.dot(a, b, preferred_element_type=_F32)


def _msg_kernel(coeff_ref, adjT_ref, h0_ref, bases_ref, bias_ref, out_ref,
                h_s, hcat_s, adj2_s, pcat_s, whi_s, wlo_s, acc):
    l = pl.program_id(0)
    r = pl.program_id(1)
    bi = pl.program_id(2)

    @pl.when((l == 0) & (r == 0))
    def _():
        h_s[bi] = h0_ref[bi]
        a = adjT_ref[bi]
        adj2_s[bi, :, :_N] = a
        adj2_s[bi, :, _N:] = a

    @pl.when(r == 0)
    def _():
        hi, lo = _split(h_s[bi])
        hcat_s[bi, :_N, :] = hi
        hcat_s[bi, _N:, :] = lo
        acc[bi] = jnp.zeros_like(acc[bi])

    @pl.when(bi == 0)
    def _():
        w = coeff_ref[l, r + 1, 0] * bases_ref[0, 0]
        for k in range(1, _R):
            w += coeff_ref[l, r + 1, k] * bases_ref[0, k]
        hi, lo = _split(w)
        whi_s[...] = hi
        wlo_s[...] = lo

    t = _bdot(hcat_s[bi], whi_s[...])
    proj = t[:_N] + t[_N:] + _bdot(hcat_s[bi, :_N, :], wlo_s[...])
    phi, plo = _split(proj)
    pcat_s[:_N, :] = phi
    pcat_s[_N:, :] = plo
    mtc = (adj2_s[bi] == (r + 1)).astype(_BF16)
    acc[bi] += _bdot(mtc, pcat_s[...])

    @pl.when(r == _NR - 1)
    def _():
        h_s[bi] = acc[bi] + bias_ref[l][None, :]

    @pl.when((l == _L - 1) & (r == _NR - 1))
    def _():
        out_ref[bi] = acc[bi] + bias_ref[l][None, :]


def kernel(inputs_embeds, attention_mask, discourse_adj, coeff, bases, bias):
    del attention_mask

    adjT = jnp.swapaxes(discourse_adj, 1, 2)

    out = pl.pallas_call(
        _msg_kernel,
        grid=(_L, _NR, _B),
        in_specs=[
            pl.BlockSpec(memory_space=pltpu.SMEM),
            pl.BlockSpec((_B, _N, _N), lambda l, r, bi: (0, 0, 0)),
            pl.BlockSpec((_B, _N, _D), lambda l, r, bi: (0, 0, 0)),
            pl.BlockSpec((1, _R, _D, _D), lambda l, r, bi: (l, 0, 0, 0)),
            pl.BlockSpec((_L, _D), lambda l, r, bi: (0, 0)),
        ],
        out_specs=pl.BlockSpec((_B, _N, _D), lambda l, r, bi: (0, 0, 0)),
        out_shape=jax.ShapeDtypeStruct((_B, _N, _D), _F32),
        scratch_shapes=[
            pltpu.VMEM((_B, _N, _D), _F32),
            pltpu.VMEM((_B, 2 * _N, _D), _BF16),
            pltpu.VMEM((_B, _N, 2 * _N), jnp.int32),
            pltpu.VMEM((2 * _N, _D), _BF16),
            pltpu.VMEM((_D, _D), _BF16),
            pltpu.VMEM((_D, _D), _BF16),
            pltpu.VMEM((_B, _N, _D), _F32),
        ],
    )(coeff, adjT, inputs_embeds, bases, bias)
    return out


# speedup vs baseline: 92.8592x; 1.3088x over previous
import jax
import jax.numpy as jnp
from jax.experimental import pallas as pl
from jax.experimental.pallas import tpu as pltpu

_B, _N, _D = 4, 256, 512
_R, _L = 17, 3
_NR = _R - 1

_F32 = jnp.float32
_BF16 = jnp.bfloat16


def _split(x):
    hi = x.astype(_BF16)
    lo = (x - hi.astype(_F32)).astype(_BF16)
    return hi, lo


def _bdot(a, b):
    return jnp.dot(a, b, preferred_element_type=_F32)


def _msg_kernel(coeff_ref, adjT_ref, h0_ref, bases_ref, bias_ref, out_ref,
                h_s, hcat_s, whi_s, wlo_s, wcat_s, acc):
    l = pl.program_id(0)
    r = pl.program_id(1)

    @pl.when((l == 0) & (r == 0))
    def _():
        h_s[...] = h0_ref[...]

    @pl.when(r == 0)
    def _():
        for bi in range(_B):
            hi, lo = _split(h_s[bi])
            hcat_s[bi * _N:(bi + 1) * _N, :_D] = hi
            hcat_s[bi * _N:(bi + 1) * _N, _D:] = lo
        acc[...] = jnp.zeros_like(acc[...])

    w = coeff_ref[l, r + 1, 0] * bases_ref[0, 0]
    for k in range(1, _R):
        w += coeff_ref[l, r + 1, k] * bases_ref[0, k]
    whi, wlo = _split(w)
    whi_s[...] = whi
    wlo_s[...] = wlo
    wcat_s[:_D, :] = whi
    wcat_s[_D:, :] = whi

    proj = (_bdot(hcat_s[...], wcat_s[...])
            + _bdot(hcat_s[:, :_D], wlo_s[...]))

    for bi in range(_B):
        p = proj[bi * _N:(bi + 1) * _N, :]
        phi, plo = _split(p)
        pcat = jnp.concatenate([phi, plo], axis=0)
        mt = (adjT_ref[bi] == (r + 1)).astype(_BF16)
        mtc = jnp.concatenate([mt, mt], axis=1)
        acc[bi] += _bdot(mtc, pcat)

    @pl.when(r == _NR - 1)
    def _():
        h_s[...] = acc[...] + bias_ref[l][None, None, :]

    @pl.when((l == _L - 1) & (r == _NR - 1))
    def _():
        out_ref[...] = acc[...] + bias_ref[l][None, None, :]


def kernel(inputs_embeds, attention_mask, discourse_adj, coeff, bases, bias):
    del attention_mask

    adjT = jnp.swapaxes(discourse_adj, 1, 2)

    out = pl.pallas_call(
        _msg_kernel,
        grid=(_L, _NR),
        in_specs=[
            pl.BlockSpec(memory_space=pltpu.SMEM),
            pl.BlockSpec((_B, _N, _N), lambda l, r: (0, 0, 0)),
            pl.BlockSpec((_B, _N, _D), lambda l, r: (0, 0, 0)),
            pl.BlockSpec((1, _R, _D, _D), lambda l, r: (l, 0, 0, 0)),
            pl.BlockSpec((_L, _D), lambda l, r: (0, 0)),
        ],
        out_specs=pl.BlockSpec((_B, _N, _D), lambda l, r: (0, 0, 0)),
        out_shape=jax.ShapeDtypeStruct((_B, _N, _D), _F32),
        scratch_shapes=[
            pltpu.VMEM((_B, _N, _D), _F32),
            pltpu.VMEM((_B * _N, 2 * _D), _BF16),
            pltpu.VMEM((_D, _D), _BF16),
            pltpu.VMEM((_D, _D), _BF16),
            pltpu.VMEM((2 * _D, _D), _BF16),
            pltpu.VMEM((_B, _N, _D), _F32),
        ],
    )(coeff, adjT, inputs_embeds, bases, bias)
    return out


# speedup vs baseline: 93.4184x; 1.0060x over previous
import jax
import jax.numpy as jnp
from jax.experimental import pallas as pl
from jax.experimental.pallas import tpu as pltpu

_B, _N, _D = 4, 256, 512
_R, _L = 17, 3
_NR = _R - 1

_F32 = jnp.float32
_BF16 = jnp.bfloat16


def _split(x):
    hi = x.astype(_BF16)
    lo = (x - hi.astype(_F32)).astype(_BF16)
    return hi, lo


def _fdot(a, b):
    return jnp.dot(a, b, preferred_element_type=_F32)


def _build_w(coeff_ref, bases_ref, wstack_s, l, r, buf):
    w = coeff_ref[l, r + 1, 0] * bases_ref[0, 0]
    for k in range(1, _R):
        w += coeff_ref[l, r + 1, k] * bases_ref[0, k]
    hi, lo = _split(w)
    wstack_s[buf, :_D, :] = hi
    wstack_s[buf, _D:2 * _D, :] = hi
    wstack_s[buf, 2 * _D:, :] = lo


def _msg_kernel(coeff_ref, adjT_ref, h0_ref, bases_ref, bias_ref, out_ref,
                h_s, hcat_s, wstack_s, acc):
    l = pl.program_id(0)
    r = pl.program_id(1)

    @pl.when((l == 0) & (r == 0))
    def _():
        h_s[...] = h0_ref[...]

    @pl.when(r == 0)
    def _():
        for bi in range(_B):
            hi, lo = _split(h_s[bi])
            hcat_s[bi * _N:(bi + 1) * _N, :_D] = hi
            hcat_s[bi * _N:(bi + 1) * _N, _D:2 * _D] = lo
            hcat_s[bi * _N:(bi + 1) * _N, 2 * _D:] = hi
        acc[...] = jnp.zeros_like(acc[...])
        _build_w(coeff_ref, bases_ref, wstack_s, l, 0, 0)

    @pl.when(r < _NR - 1)
    def _():
        _build_w(coeff_ref, bases_ref, wstack_s, l, r + 1, (r + 1) % 2)

    proj = _fdot(hcat_s[...], wstack_s[r % 2])

    for bi in range(_B):
        p = proj[bi * _N:(bi + 1) * _N, :]
        phi, plo = _split(p)
        pcat = jnp.concatenate([phi, plo], axis=0)
        mt = (adjT_ref[bi] == (r + 1)).astype(_BF16)
        mtc = jnp.concatenate([mt, mt], axis=1)
        acc[bi] += _fdot(mtc, pcat)

    @pl.when(r == _NR - 1)
    def _():
        h_s[...] = acc[...] + bias_ref[l][None, None, :]

    @pl.when((l == _L - 1) & (r == _NR - 1))
    def _():
        out_ref[...] = acc[...] + bias_ref[l][None, None, :]


def kernel(inputs_embeds, attention_mask, discourse_adj, coeff, bases, bias):
    del attention_mask

    adjT = jnp.swapaxes(discourse_adj, 1, 2)

    out = pl.pallas_call(
        _msg_kernel,
        grid=(_L, _NR),
        in_specs=[
            pl.BlockSpec(memory_space=pltpu.SMEM),
            pl.BlockSpec((_B, _N, _N), lambda l, r: (0, 0, 0)),
            pl.BlockSpec((_B, _N, _D), lambda l, r: (0, 0, 0)),
            pl.BlockSpec((1, _R, _D, _D), lambda l, r: (l, 0, 0, 0)),
            pl.BlockSpec((_L, _D), lambda l, r: (0, 0)),
        ],
        out_specs=pl.BlockSpec((_B, _N, _D), lambda l, r: (0, 0, 0)),
        out_shape=jax.ShapeDtypeStruct((_B, _N, _D), _F32),
        scratch_shapes=[
            pltpu.VMEM((_B, _N, _D), _F32),
            pltpu.VMEM((_B * _N, 3 * _D), _BF16),
            pltpu.VMEM((2, 3 * _D, _D), _BF16),
            pltpu.VMEM((_B, _N, _D), _F32),
        ],
    )(coeff, adjT, inputs_embeds, bases, bias)
    return out


# speedup vs baseline: 96.1600x; 1.0293x over previous
import jax
import jax.numpy as jnp
from jax.experimental import pallas as pl
from jax.experimental.pallas import tpu as pltpu

_B, _N, _D = 4, 256, 512
_R, _L = 17, 3
_NR = _R - 1

_F32 = jnp.float32
_BF16 = jnp.bfloat16


def _split(x):
    hi = x.astype(_BF16)
    lo = (x - hi.astype(_F32)).astype(_BF16)
    return hi, lo


def _fdot(a, b):
    return jnp.dot(a, b, preferred_element_type=_F32)


def _build_w(coeff_ref, bases_ref, wstack_s, l, r, buf):
    w = coeff_ref[l, r + 1, 0] * bases_ref[0, 0]
    for k in range(1, _R):
        w += coeff_ref[l, r + 1, k] * bases_ref[0, k]
    hi, lo = _split(w)
    wstack_s[buf, :_D, :] = hi
    wstack_s[buf, _D:2 * _D, :] = hi
    wstack_s[buf, 2 * _D:, :] = lo


def _msg_kernel(coeff_ref, adjT_ref, h0_ref, bases_ref, bias_ref, out_ref,
                h_s, hcat_s, wstack_s, acc):
    l = pl.program_id(0)
    r = pl.program_id(1)

    @pl.when((l == 0) & (r == 0))
    def _():
        h_s[...] = h0_ref[...]

    @pl.when(r == 0)
    def _():
        for bi in range(_B):
            hi, lo = _split(h_s[bi])
            hcat_s[bi * _N:(bi + 1) * _N, :_D] = hi
            hcat_s[bi * _N:(bi + 1) * _N, _D:2 * _D] = lo
            hcat_s[bi * _N:(bi + 1) * _N, 2 * _D:] = hi
        acc[...] = jnp.zeros_like(acc[...])
        _build_w(coeff_ref, bases_ref, wstack_s, l, 0, 0)

    rb = jnp.minimum(r + 1, _NR - 1)
    _build_w(coeff_ref, bases_ref, wstack_s, l, rb, (r + 1) % 2)

    proj = _fdot(hcat_s[...], wstack_s[r % 2])

    for bi in range(_B):
        p = proj[bi * _N:(bi + 1) * _N, :]
        phi, plo = _split(p)
        pcat = jnp.concatenate([phi, plo], axis=0)
        mt = (adjT_ref[bi] == (r + 1)).astype(_BF16)
        mtc = jnp.concatenate([mt, mt], axis=1)
        acc[bi] += _fdot(mtc, pcat)

    @pl.when(r == _NR - 1)
    def _():
        h_s[...] = acc[...] + bias_ref[l][None, None, :]

    @pl.when((l == _L - 1) & (r == _NR - 1))
    def _():
        out_ref[...] = acc[...] + bias_ref[l][None, None, :]


def kernel(inputs_embeds, attention_mask, discourse_adj, coeff, bases, bias):
    del attention_mask

    adjT = jnp.swapaxes(discourse_adj, 1, 2)

    out = pl.pallas_call(
        _msg_kernel,
        grid=(_L, _NR),
        in_specs=[
            pl.BlockSpec(memory_space=pltpu.SMEM),
            pl.BlockSpec((_B, _N, _N), lambda l, r: (0, 0, 0)),
            pl.BlockSpec((_B, _N, _D), lambda l, r: (0, 0, 0)),
            pl.BlockSpec((1, _R, _D, _D), lambda l, r: (l, 0, 0, 0)),
            pl.BlockSpec((_L, _D), lambda l, r: (0, 0)),
        ],
        out_specs=pl.BlockSpec((_B, _N, _D), lambda l, r: (0, 0, 0)),
        out_shape=jax.ShapeDtypeStruct((_B, _N, _D), _F32),
        scratch_shapes=[
            pltpu.VMEM((_B, _N, _D), _F32),
            pltpu.VMEM((_B * _N, 3 * _D), _BF16),
            pltpu.VMEM((2, 3 * _D, _D), _BF16),
            pltpu.VMEM((_B, _N, _D), _F32),
        ],
    )(coeff, adjT, inputs_embeds, bases, bias)
    return out
